# baseline (speedup 1.0000x reference)
import jax
import jax.numpy as jnp
from jax import lax
from jax.experimental import pallas as pl
from jax.experimental.pallas import tpu as pltpu
from jax.experimental.pallas import tpu_sc as plsc

B = 4096
F = 26
D = 32
NC = 2
NS = 16
NV = B // 16
CH = 1024
NCH = 98
NROWS = 2600000
TAIL0 = NROWS - 64
TAILN = 64
CARD = 100000
NBUCK = NCH + 1
CAP = 128
POOLSZ = NBUCK * CAP
CURSZ = 112
NRING = 32
HW = 16 * D
DUMP = F * B * D


def _scalar(x):
    return x if x.ndim == 0 else x[0]


def _body(xT_hbm, tbl_hbm, bias_hbm, offs_hbm, tail_hbm, out_hbm,
          x_v, ridx_v, bufA0, bufA1, bufA2, bufA3, bufB0, bufB1, bufB2,
          bufB3, tail_v, bias_v, offs_v, pool_v, cur_v,
          slab_v, semA, semB, semO):
    bufsA = (bufA0, bufA1, bufA2, bufA3)
    bufsB = (bufB0, bufB1, bufB2, bufB3)
    wid = lax.axis_index("s") * NC + lax.axis_index("c")
    lane = lax.iota(jnp.int32, 16)

    @pl.when(wid < F)
    def _work():
        base = pl.multiple_of(wid * B, 8)
        pltpu.sync_copy(xT_hbm.at[pl.ds(base, B)], x_v)
        pltpu.sync_copy(bias_hbm.at[pl.ds(pl.multiple_of(wid * D, 8), D)],
                        bias_v)
        pltpu.sync_copy(offs_hbm, offs_v)
        pltpu.sync_copy(tail_hbm, tail_v)

        o0 = offs_v[pl.ds(0, 16)]
        o1 = offs_v[pl.ds(16, 16)]
        off_f = (jnp.sum(jnp.where(lane == wid, o0, 0))
                 + jnp.sum(jnp.where(lane + 16 == wid, o1, 0)))
        row_lo = (off_f // 128) * 128
        hi_f = jnp.minimum(((off_f + CARD + 127) // 128) * 128, TAIL0)

        def fire(c, bufs, sem):
            start = pl.multiple_of(
                jnp.minimum(row_lo + c * CH, hi_f - CH), 128)
            for g in range(4):
                pltpu.async_copy(tbl_hbm.at[g, :, pl.ds(start, CH)],
                                 bufs[g], sem)

        def drain(bufs, sem):
            for g in range(4):
                pltpu.make_async_copy(
                    tbl_hbm.at[0, :, pl.ds(0, CH)], bufs[g], sem).wait()

        fire(0, bufsA, semA)
        fire(1, bufsB, semB)

        b_lo = bias_v[pl.ds(0, 16)]
        b_hi = bias_v[pl.ds(16, 16)]
        off_bc = jnp.zeros((16,), jnp.int32) + off_f

        for t in range(CURSZ // 16):
            cur_v[pl.ds(t * 16, 16)] = jnp.zeros((16,), jnp.int32)

        def mk_idx(v, _):
            sl = pl.ds(v * 16, 16)
            ridx_v[sl] = x_v[sl] + off_bc
            return ()

        lax.fori_loop(0, NV, mk_idx, ())

        def bucket(v, _):
            rv = ridx_v[pl.ds(v * 16, 16)]
            cid = jnp.where(rv >= TAIL0, NCH,
                            lax.shift_right_arithmetic(rv - row_lo, 10))
            posn = lane + v * 16

            def wcond(pend):
                return _scalar(plsc.all_reduce_population_count(pend)) > 0

            def wbody(pend):
                cur = plsc.load_gather(cur_v, [cid])
                slot = jnp.minimum(cid * CAP + cur, POOLSZ - 1)
                plsc.store_scatter(pool_v, [slot], posn, mask=pend)
                back = plsc.load_gather(pool_v, [slot])
                ok = pend & ((back == posn) | (cur >= CAP))
                plsc.store_scatter(cur_v, [cid], cur + 1,
                                   mask=ok & (cur < CAP))
                return pend & ~ok

            lax.while_loop(wcond, wbody, lane == lane)
            return ()

        lax.fori_loop(0, NV, bucket, ())

        def process(c, bstart, gather_fn, jh0):
            c16 = (c // 16) * 16
            nv16 = cur_v[pl.ds(c16, 16)]
            n = jnp.sum(jnp.where(lane == c - c16, nv16, 0))
            nh = (n + 15) >> 4

            def ext(t, jh):
                posv = pool_v[pl.ds(c * CAP + t * 16, 16)]
                valid = (t * 16 + lane) < n
                rv = plsc.load_gather(ridx_v, [posv], mask=valid)
                rloc = rv - bstart
                hbase = lax.rem(jh, NRING) * HW

                @pl.when(jh >= NRING)
                def _():
                    pltpu.make_async_copy(
                        tbl_hbm.at[0, 0, pl.ds(0, HW)],
                        slab_v.at[pl.ds(0, HW)], semO).wait()

                for d in range(D):
                    vals = gather_fn(d, rloc, valid)
                    bd = b_lo[d] if d < 16 else b_hi[d - 16]
                    plsc.store_scatter(slab_v, [hbase + lane * D + d],
                                       vals + bd, mask=valid)
                for k in range(16):
                    ok = (t * 16 + k) < n
                    ooff = jnp.where(ok, (wid * B + posv[k]) * D, DUMP)
                    pltpu.async_copy(
                        slab_v.at[pl.ds(hbase + k * D, D)],
                        out_hbm.at[pl.ds(pl.multiple_of(ooff, 8), D)],
                        semO)
                return jh + 1

            return lax.fori_loop(0, nh, ext, jh0)

        def mk_gather(bufs):
            def g_fn(d, rloc, valid):
                srow = jnp.zeros((16,), jnp.int32) + (d % 8)
                return plsc.load_gather(bufs[d // 8], [srow, rloc],
                                        mask=valid)
            return g_fn

        def tail_gather(d, rloc, valid):
            return plsc.load_gather(tail_v, [rloc + d * TAILN], mask=valid)

        def two_chunks(i, jh):
            c0 = 2 * i
            drain(bufsA, semA)
            jh = process(c0, jnp.minimum(row_lo + c0 * CH, hi_f - CH),
                         mk_gather(bufsA), jh)
            fire(jnp.minimum(c0 + 2, NCH - 1), bufsA, semA)
            c1 = c0 + 1
            drain(bufsB, semB)
            jh = process(c1, jnp.minimum(row_lo + c1 * CH, hi_f - CH),
                         mk_gather(bufsB), jh)
            fire(jnp.minimum(c1 + 2, NCH - 1), bufsB, semB)
            return jh

        jh = lax.fori_loop(0, NCH // 2, two_chunks, 0)
        drain(bufsA, semA)
        drain(bufsB, semB)

        jh = process(NCH, TAIL0, tail_gather, jh)

        def ring_drain(_, __):
            pltpu.make_async_copy(
                tbl_hbm.at[0, 0, pl.ds(0, HW)],
                slab_v.at[pl.ds(0, HW)], semO).wait()
            return ()

        lax.fori_loop(0, jnp.minimum(jh, NRING), ring_drain, ())


@jax.jit
def _cat_embeddings(xT, tbl3, bias1, offs_pad, tail64):
    mesh = plsc.VectorSubcoreMesh(core_axis_name="c", subcore_axis_name="s")
    kern = pl.kernel(
        _body,
        out_type=jax.ShapeDtypeStruct((F * B * D + D,), jnp.float32),
        mesh=mesh,
        scratch_types=[
            pltpu.VMEM((B,), jnp.int32),
            pltpu.VMEM((B,), jnp.int32),
            pltpu.VMEM((8, CH), jnp.float32),
            pltpu.VMEM((8, CH), jnp.float32),
            pltpu.VMEM((8, CH), jnp.float32),
            pltpu.VMEM((8, CH), jnp.float32),
            pltpu.VMEM((8, CH), jnp.float32),
            pltpu.VMEM((8, CH), jnp.float32),
            pltpu.VMEM((8, CH), jnp.float32),
            pltpu.VMEM((8, CH), jnp.float32),
            pltpu.VMEM((D * TAILN,), jnp.float32),
            pltpu.VMEM((D,), jnp.float32),
            pltpu.VMEM((32,), jnp.int32),
            pltpu.VMEM((POOLSZ,), jnp.int32),
            pltpu.VMEM((CURSZ,), jnp.int32),
            pltpu.VMEM((NRING * HW,), jnp.float32),
            pltpu.SemaphoreType.DMA,
            pltpu.SemaphoreType.DMA,
            pltpu.SemaphoreType.DMA,
        ],
        compiler_params=pltpu.CompilerParams(needs_layout_passes=False),
    )
    return kern(xT, tbl3, bias1, offs_pad, tail64)


def kernel(x, table, bias, offsets):
    xT = x.astype(jnp.int32).T.reshape(F * B)
    tbl3 = table.T.reshape(4, 8, NROWS)
    bias1 = bias.reshape(F * D)
    offs_pad = jnp.zeros((32,), jnp.int32).at[:F].set(offsets.astype(jnp.int32))
    tail64 = table[TAIL0:].T.reshape(D * TAILN)
    out = _cat_embeddings(xT, tbl3, bias1, offs_pad, tail64)
    return out[:DUMP].reshape(F, B, D).transpose(1, 0, 2)


# speedup vs baseline: 1.3457x; 1.3457x over previous
import jax
import jax.numpy as jnp
from jax import lax
from jax.experimental import pallas as pl
from jax.experimental.pallas import tpu as pltpu
from jax.experimental.pallas import tpu_sc as plsc

B = 4096
F = 26
D = 32
NC = 2
NS = 16
NW = NC * NS
NV = 2 * B // 16
CH = 1024
RANGE = 81280
NCH = 80
NROWS = 2600000
TAIL0 = NROWS - 64
TAILN = 64
NBUCK = NCH + 1
CAP = 128
POOLSZ = NBUCK * CAP
CURSZ = 96
NRING = 32
HW = 16 * D
DUMP = F * B * D


def _scalar(x):
    return x if x.ndim == 0 else x[0]


def _body(xT_hbm, tbl_hbm, bias_hbm, offs_hbm, tail_hbm, out_hbm,
          x_v, ridx_v, bufA0, bufA1, bufA2, bufA3, bufB0, bufB1, bufB2,
          bufB3, tail_v, bias_v, offs_v, pool_v, cur_v,
          slab_v, semA, semB, semO):
    bufsA = (bufA0, bufA1, bufA2, bufA3)
    bufsB = (bufB0, bufB1, bufB2, bufB3)
    wid = lax.axis_index("s") * NC + lax.axis_index("c")
    lane = lax.iota(jnp.int32, 16)

    range_lo = wid * RANGE
    range_hi = jnp.minimum(range_lo + RANGE, NROWS)
    stream_hi = jnp.minimum(range_hi, TAIL0)

    pltpu.sync_copy(offs_hbm, offs_v)
    pltpu.sync_copy(tail_hbm, tail_v)
    o0 = offs_v[pl.ds(0, 16)]
    o1 = offs_v[pl.ds(16, 16)]

    def field_of(row):
        return (_scalar(plsc.all_reduce_population_count(o0 <= row))
                + _scalar(plsc.all_reduce_population_count(o1 <= row)) - 1)

    f0 = field_of(range_lo)
    f1 = field_of(range_hi - 1)
    dual = f1 > f0

    def off_of(f):
        return (jnp.sum(jnp.where(lane == f, o0, 0))
                + jnp.sum(jnp.where(lane + 16 == f, o1, 0)))

    off0 = off_of(f0)
    off1 = off_of(f1)

    pltpu.sync_copy(xT_hbm.at[pl.ds(pl.multiple_of(f0 * B, 8), B)],
                    x_v.at[pl.ds(0, B)])
    pltpu.sync_copy(xT_hbm.at[pl.ds(pl.multiple_of(f1 * B, 8), B)],
                    x_v.at[pl.ds(B, B)])
    pltpu.sync_copy(bias_hbm.at[pl.ds(pl.multiple_of(f0 * D, 8), D)],
                    bias_v.at[pl.ds(0, D)])
    pltpu.sync_copy(bias_hbm.at[pl.ds(pl.multiple_of(f1 * D, 8), D)],
                    bias_v.at[pl.ds(D, D)])

    def fire(c, bufs, sem):
        start = pl.multiple_of(
            jnp.minimum(range_lo + c * CH, stream_hi - CH), 128)
        for g in range(4):
            pltpu.async_copy(tbl_hbm.at[g, :, pl.ds(start, CH)],
                             bufs[g], sem)

    def drain(bufs, sem):
        for g in range(4):
            pltpu.make_async_copy(
                tbl_hbm.at[0, :, pl.ds(0, CH)], bufs[g], sem).wait()

    fire(0, bufsA, semA)
    fire(1, bufsB, semB)

    for t in range(CURSZ // 16):
        cur_v[pl.ds(t * 16, 16)] = jnp.zeros((16,), jnp.int32)

    def mk_idx(v, _):
        sl = pl.ds(v * 16, 16)
        offh = jnp.where(v < NV // 2, off0, off1)
        ridx_v[sl] = x_v[sl] + (jnp.zeros((16,), jnp.int32) + offh)
        return ()

    lax.fori_loop(0, NV, mk_idx, ())

    def bucket(v, _):
        rv = ridx_v[pl.ds(v * 16, 16)]
        cid = jnp.where(rv >= TAIL0, NCH,
                        lax.shift_right_arithmetic(rv - range_lo, 10))
        cid = jnp.clip(cid, 0, NCH)
        posbase = jnp.where(v < NV // 2, f0 * B, f1 * B - B)
        posn = posbase + lane + v * 16
        okh = jnp.where(v < NV // 2, True, dual)
        pend0 = (rv >= range_lo) & (rv < range_hi) & okh

        def wcond(pend):
            return _scalar(plsc.all_reduce_population_count(pend)) > 0

        def wbody(pend):
            cur = plsc.load_gather(cur_v, [cid], mask=pend)
            slot = jnp.minimum(cid * CAP + cur, POOLSZ - 1)
            plsc.store_scatter(pool_v, [slot], posn, mask=pend)
            back = plsc.load_gather(pool_v, [slot], mask=pend)
            ok = pend & ((back == posn) | (cur >= CAP))
            plsc.store_scatter(cur_v, [cid], cur + 1,
                               mask=ok & (cur < CAP))
            return pend & ~ok

        lax.while_loop(wcond, wbody, pend0)
        return ()

    lax.fori_loop(0, NV, bucket, ())

    def process(c, bstart, gather_fn, jh0):
        c16 = (c // 16) * 16
        nv16 = cur_v[pl.ds(c16, 16)]
        n = jnp.sum(jnp.where(lane == c - c16, nv16, 0))
        nh = (n + 15) >> 4

        def ext(t, jh):
            posv = pool_v[pl.ds(c * CAP + t * 16, 16)]
            valid = (t * 16 + lane) < n
            rv = plsc.load_gather(ridx_v, [posv - f0 * B], mask=valid)
            rloc = rv - bstart
            fidx = (lax.shift_right_arithmetic(posv, 12) - f0) * D
            hbase = lax.rem(jh, NRING) * HW

            @pl.when(jh >= NRING)
            def _():
                pltpu.make_async_copy(
                    tbl_hbm.at[0, 0, pl.ds(0, HW)],
                    slab_v.at[pl.ds(0, HW)], semO).wait()

            for d in range(D):
                vals = gather_fn(d, rloc, valid)
                bvals = plsc.load_gather(bias_v, [fidx + d], mask=valid)
                plsc.store_scatter(slab_v, [hbase + lane * D + d],
                                   vals + bvals, mask=valid)
            for k in range(16):
                ok = (t * 16 + k) < n
                ooff = jnp.where(ok, posv[k] * D, DUMP + k * D)
                pltpu.async_copy(
                    slab_v.at[pl.ds(hbase + k * D, D)],
                    out_hbm.at[pl.ds(pl.multiple_of(ooff, 8), D)],
                    semO)
            return jh + 1

        return lax.fori_loop(0, nh, ext, jh0)

    def mk_gather(bufs):
        def g_fn(d, rloc, valid):
            srow = jnp.zeros((16,), jnp.int32) + (d % 8)
            return plsc.load_gather(bufs[d // 8], [srow, rloc], mask=valid)
        return g_fn

    def tail_gather(d, rloc, valid):
        return plsc.load_gather(tail_v, [rloc + d * TAILN], mask=valid)

    def two_chunks(i, jh):
        c0 = 2 * i
        drain(bufsA, semA)
        jh = process(c0, jnp.minimum(range_lo + c0 * CH, stream_hi - CH),
                     mk_gather(bufsA), jh)
        fire(jnp.minimum(c0 + 2, NCH - 1), bufsA, semA)
        c1 = c0 + 1
        drain(bufsB, semB)
        jh = process(c1, jnp.minimum(range_lo + c1 * CH, stream_hi - CH),
                     mk_gather(bufsB), jh)
        fire(jnp.minimum(c1 + 2, NCH - 1), bufsB, semB)
        return jh

    jh = lax.fori_loop(0, NCH // 2, two_chunks, 0)
    drain(bufsA, semA)
    drain(bufsB, semB)

    jh = process(NCH, TAIL0, tail_gather, jh)

    def ring_drain(_, __):
        pltpu.make_async_copy(
            tbl_hbm.at[0, 0, pl.ds(0, HW)],
            slab_v.at[pl.ds(0, HW)], semO).wait()
        return ()

    lax.fori_loop(0, jnp.minimum(jh, NRING), ring_drain, ())


@jax.jit
def _cat_embeddings(xT, tbl3, bias1, offs_pad, tail64):
    mesh = plsc.VectorSubcoreMesh(core_axis_name="c", subcore_axis_name="s")
    kern = pl.kernel(
        _body,
        out_type=jax.ShapeDtypeStruct((F * B * D + 16 * D,), jnp.float32),
        mesh=mesh,
        scratch_types=[
            pltpu.VMEM((2 * B,), jnp.int32),
            pltpu.VMEM((2 * B,), jnp.int32),
            pltpu.VMEM((8, CH), jnp.float32),
            pltpu.VMEM((8, CH), jnp.float32),
            pltpu.VMEM((8, CH), jnp.float32),
            pltpu.VMEM((8, CH), jnp.float32),
            pltpu.VMEM((8, CH), jnp.float32),
            pltpu.VMEM((8, CH), jnp.float32),
            pltpu.VMEM((8, CH), jnp.float32),
            pltpu.VMEM((8, CH), jnp.float32),
            pltpu.VMEM((D * TAILN,), jnp.float32),
            pltpu.VMEM((2 * D,), jnp.float32),
            pltpu.VMEM((32,), jnp.int32),
            pltpu.VMEM((POOLSZ,), jnp.int32),
            pltpu.VMEM((CURSZ,), jnp.int32),
            pltpu.VMEM((NRING * HW,), jnp.float32),
            pltpu.SemaphoreType.DMA,
            pltpu.SemaphoreType.DMA,
            pltpu.SemaphoreType.DMA,
        ],
        compiler_params=pltpu.CompilerParams(needs_layout_passes=False),
    )
    return kern(xT, tbl3, bias1, offs_pad, tail64)


def kernel(x, table, bias, offsets):
    xT = x.astype(jnp.int32).T.reshape(F * B)
    tbl3 = table.T.reshape(4, 8, NROWS)
    bias1 = bias.reshape(F * D)
    offs_pad = jnp.full((32,), 1 << 30, jnp.int32).at[:F].set(
        offsets.astype(jnp.int32))
    tail64 = table[TAIL0:].T.reshape(D * TAILN)
    out = _cat_embeddings(xT, tbl3, bias1, offs_pad, tail64)
    return out[:DUMP].reshape(F, B, D).transpose(1, 0, 2)


# speedup vs baseline: 1.3764x; 1.0228x over previous
import jax
import jax.numpy as jnp
from jax import lax
from jax.experimental import pallas as pl
from jax.experimental.pallas import tpu as pltpu
from jax.experimental.pallas import tpu_sc as plsc

B = 4096
F = 26
D = 32
NC = 2
NS = 16
NW = NC * NS
NV = 2 * B // 16
CH = 1024
RANGE = 81280
NCH = 80
NROWS = 2600000
TAIL0 = NROWS - 64
TAILN = 64
NBUCK = NCH + 1
CAP = 128
POOLSZ = NBUCK * CAP
CURSZ = 96
NRING = 32
HW = 16 * D
DUMP = F * B * D


def _scalar(x):
    return x if x.ndim == 0 else x[0]


def _body(xT_hbm, tbl_hbm, bias_hbm, offs_hbm, tail_hbm, out_hbm,
          x_v, ridx_v, bufA0, bufA1, bufA2, bufA3, bufB0, bufB1, bufB2,
          bufB3, tail_v, bias_v, offs_v, pool_v, cur_v,
          slab_v, semA0, semA1, semA2, semA3, semB0, semB1, semB2, semB3,
          semO):
    semA = (semA0, semA1, semA2, semA3)
    semB = (semB0, semB1, semB2, semB3)
    bufsA = (bufA0, bufA1, bufA2, bufA3)
    bufsB = (bufB0, bufB1, bufB2, bufB3)
    wid = lax.axis_index("s") * NC + lax.axis_index("c")
    lane = lax.iota(jnp.int32, 16)

    range_lo = wid * RANGE
    range_hi = jnp.minimum(range_lo + RANGE, NROWS)
    stream_hi = jnp.minimum(range_hi, TAIL0)

    pltpu.sync_copy(offs_hbm, offs_v)
    pltpu.sync_copy(tail_hbm, tail_v)
    o0 = offs_v[pl.ds(0, 16)]
    o1 = offs_v[pl.ds(16, 16)]

    def field_of(row):
        return (_scalar(plsc.all_reduce_population_count(o0 <= row))
                + _scalar(plsc.all_reduce_population_count(o1 <= row)) - 1)

    f0 = field_of(range_lo)
    f1 = field_of(range_hi - 1)
    dual = f1 > f0

    def off_of(f):
        return (jnp.sum(jnp.where(lane == f, o0, 0))
                + jnp.sum(jnp.where(lane + 16 == f, o1, 0)))

    off0 = off_of(f0)
    off1 = off_of(f1)

    pltpu.sync_copy(xT_hbm.at[pl.ds(pl.multiple_of(f0 * B, 8), B)],
                    x_v.at[pl.ds(0, B)])
    pltpu.sync_copy(xT_hbm.at[pl.ds(pl.multiple_of(f1 * B, 8), B)],
                    x_v.at[pl.ds(B, B)])
    pltpu.sync_copy(bias_hbm.at[pl.ds(pl.multiple_of(f0 * D, 8), D)],
                    bias_v.at[pl.ds(0, D)])
    pltpu.sync_copy(bias_hbm.at[pl.ds(pl.multiple_of(f1 * D, 8), D)],
                    bias_v.at[pl.ds(D, D)])

    def fire(c, bufs, sem):
        start = pl.multiple_of(
            jnp.minimum(range_lo + c * CH, stream_hi - CH), 128)
        for g in range(4):
            pltpu.async_copy(tbl_hbm.at[g, :, pl.ds(start, CH)],
                             bufs[g], sem[g])

    def drain(bufs, sem):
        for g in range(4):
            pltpu.make_async_copy(
                tbl_hbm.at[0, :, pl.ds(0, CH)], bufs[g], sem[g]).wait()

    fire(0, bufsA, semA)
    fire(1, bufsB, semB)

    for t in range(CURSZ // 16):
        cur_v[pl.ds(t * 16, 16)] = jnp.zeros((16,), jnp.int32)

    def mk_idx(v, _):
        sl = pl.ds(v * 16, 16)
        offh = jnp.where(v < NV // 2, off0, off1)
        ridx_v[sl] = x_v[sl] + (jnp.zeros((16,), jnp.int32) + offh)
        return ()

    lax.fori_loop(0, NV, mk_idx, ())

    def bucket(v, _):
        rv = ridx_v[pl.ds(v * 16, 16)]
        cid = jnp.where(rv >= TAIL0, NCH,
                        lax.shift_right_arithmetic(rv - range_lo, 10))
        cid = jnp.clip(cid, 0, NCH)
        posbase = jnp.where(v < NV // 2, f0 * B, f1 * B - B)
        posn = posbase + lane + v * 16
        okh = jnp.where(v < NV // 2, True, dual)
        pend0 = (rv >= range_lo) & (rv < range_hi) & okh

        def wcond(pend):
            return _scalar(plsc.all_reduce_population_count(pend)) > 0

        def wbody(pend):
            cur = plsc.load_gather(cur_v, [cid], mask=pend)
            slot = jnp.minimum(cid * CAP + cur, POOLSZ - 1)
            plsc.store_scatter(pool_v, [slot], posn, mask=pend)
            back = plsc.load_gather(pool_v, [slot], mask=pend)
            ok = pend & ((back == posn) | (cur >= CAP))
            plsc.store_scatter(cur_v, [cid], cur + 1,
                               mask=ok & (cur < CAP))
            return pend & ~ok

        lax.while_loop(wcond, wbody, pend0)
        return ()

    lax.fori_loop(0, NV, bucket, ())

    def process(c, bstart, gather_fn, jh0):
        c16 = (c // 16) * 16
        nv16 = cur_v[pl.ds(c16, 16)]
        n = jnp.sum(jnp.where(lane == c - c16, nv16, 0))
        nh = (n + 15) >> 4

        def ext(t, jh):
            posv = pool_v[pl.ds(c * CAP + t * 16, 16)]
            valid = (t * 16 + lane) < n
            rv = plsc.load_gather(ridx_v, [posv - f0 * B], mask=valid)
            rloc = rv - bstart
            fidx = (lax.shift_right_arithmetic(posv, 12) - f0) * D
            hbase = lax.rem(jh, NRING) * HW

            @pl.when(jh >= NRING)
            def _():
                pltpu.make_async_copy(
                    tbl_hbm.at[0, 0, pl.ds(0, HW)],
                    slab_v.at[pl.ds(0, HW)], semO).wait()

            for d in range(D):
                vals = gather_fn(d, rloc, valid)
                bvals = plsc.load_gather(bias_v, [fidx + d], mask=valid)
                plsc.store_scatter(slab_v, [hbase + lane * D + d],
                                   vals + bvals, mask=valid)
            for k in range(16):
                ok = (t * 16 + k) < n
                ooff = jnp.where(ok, posv[k] * D, DUMP + k * D)
                pltpu.async_copy(
                    slab_v.at[pl.ds(hbase + k * D, D)],
                    out_hbm.at[pl.ds(pl.multiple_of(ooff, 8), D)],
                    semO)
            return jh + 1

        return lax.fori_loop(0, nh, ext, jh0)

    def mk_gather(bufs):
        def g_fn(d, rloc, valid):
            srow = jnp.zeros((16,), jnp.int32) + (d % 8)
            return plsc.load_gather(bufs[d // 8], [srow, rloc], mask=valid)
        return g_fn

    def tail_gather(d, rloc, valid):
        return plsc.load_gather(tail_v, [rloc + d * TAILN], mask=valid)

    def two_chunks(i, jh):
        c0 = 2 * i
        drain(bufsA, semA)
        jh = process(c0, jnp.minimum(range_lo + c0 * CH, stream_hi - CH),
                     mk_gather(bufsA), jh)
        fire(jnp.minimum(c0 + 2, NCH - 1), bufsA, semA)
        c1 = c0 + 1
        drain(bufsB, semB)
        jh = process(c1, jnp.minimum(range_lo + c1 * CH, stream_hi - CH),
                     mk_gather(bufsB), jh)
        fire(jnp.minimum(c1 + 2, NCH - 1), bufsB, semB)
        return jh

    jh = lax.fori_loop(0, NCH // 2, two_chunks, 0)
    drain(bufsA, semA)
    drain(bufsB, semB)

    jh = process(NCH, TAIL0, tail_gather, jh)

    def ring_drain(_, __):
        pltpu.make_async_copy(
            tbl_hbm.at[0, 0, pl.ds(0, HW)],
            slab_v.at[pl.ds(0, HW)], semO).wait()
        return ()

    lax.fori_loop(0, jnp.minimum(jh, NRING), ring_drain, ())


@jax.jit
def _cat_embeddings(xT, tbl3, bias1, offs_pad, tail64):
    mesh = plsc.VectorSubcoreMesh(core_axis_name="c", subcore_axis_name="s")
    kern = pl.kernel(
        _body,
        out_type=jax.ShapeDtypeStruct((F * B * D + 16 * D,), jnp.float32),
        mesh=mesh,
        scratch_types=[
            pltpu.VMEM((2 * B,), jnp.int32),
            pltpu.VMEM((2 * B,), jnp.int32),
            pltpu.VMEM((8, CH), jnp.float32),
            pltpu.VMEM((8, CH), jnp.float32),
            pltpu.VMEM((8, CH), jnp.float32),
            pltpu.VMEM((8, CH), jnp.float32),
            pltpu.VMEM((8, CH), jnp.float32),
            pltpu.VMEM((8, CH), jnp.float32),
            pltpu.VMEM((8, CH), jnp.float32),
            pltpu.VMEM((8, CH), jnp.float32),
            pltpu.VMEM((D * TAILN,), jnp.float32),
            pltpu.VMEM((2 * D,), jnp.float32),
            pltpu.VMEM((32,), jnp.int32),
            pltpu.VMEM((POOLSZ,), jnp.int32),
            pltpu.VMEM((CURSZ,), jnp.int32),
            pltpu.VMEM((NRING * HW,), jnp.float32),
            pltpu.SemaphoreType.DMA,
            pltpu.SemaphoreType.DMA,
            pltpu.SemaphoreType.DMA,
            pltpu.SemaphoreType.DMA,
            pltpu.SemaphoreType.DMA,
            pltpu.SemaphoreType.DMA,
            pltpu.SemaphoreType.DMA,
            pltpu.SemaphoreType.DMA,
            pltpu.SemaphoreType.DMA,
        ],
        compiler_params=pltpu.CompilerParams(needs_layout_passes=False),
    )
    return kern(xT, tbl3, bias1, offs_pad, tail64)


def kernel(x, table, bias, offsets):
    xT = x.astype(jnp.int32).T.reshape(F * B)
    tbl3 = table.T.reshape(4, 8, NROWS)
    bias1 = bias.reshape(F * D)
    offs_pad = jnp.full((32,), 1 << 30, jnp.int32).at[:F].set(
        offsets.astype(jnp.int32))
    tail64 = table[TAIL0:].T.reshape(D * TAILN)
    out = _cat_embeddings(xT, tbl3, bias1, offs_pad, tail64)
    return out[:DUMP].reshape(F, B, D).transpose(1, 0, 2)
